# baseline (speedup 1.0000x reference)
import jax
import jax.numpy as jnp
import numpy as np
from jax.experimental import pallas as pl
from jax.experimental.pallas import tpu as pltpu

_SCORE_THR = 0.05
_IOU_THR = 0.5
_MAX_NUM = 100
_STRIDE = 8.0
_EPS = 1e-6
_BIG = 3.0e7
_TOPK = 8


def _argmin_idx(s, m, kf):
    return jnp.min(jnp.where(s == m, kf, _BIG))


def _allred(x, op):
    for sh in (1, 2, 4):
        x = op(x, pltpu.roll(x, sh, 0))
    for sh in (1, 2, 4, 8, 16, 32, 64):
        x = op(x, pltpu.roll(x, sh, 1))
    return x


def _nms_body(img_max, n_valid, B, cls_ref, bb_ref, px_ref, py_ref,
              out_ref, *scratch):
    skf = scratch[0]
    sb = [scratch[1 + 6 * b:1 + 6 * (b + 1)] for b in range(B)]

    shape = px_ref.shape
    row_i = jax.lax.broadcasted_iota(jnp.int32, shape, 0)
    col_i = jax.lax.broadcasted_iota(jnp.int32, shape, 1)
    kf = (row_i * 128 + col_i).astype(jnp.float32)
    rowf = row_i.astype(jnp.float32)
    skf[...] = kf
    lanef = jax.lax.broadcasted_iota(jnp.int32, (1, 128), 1).astype(jnp.float32)
    px = px_ref[...]
    py = py_ref[...]

    compact = []
    rest_max = []
    for b in range(B):
        sx1, sy1, sx2, sy2, sar, ss = sb[b]
        raw = jax.nn.sigmoid(cls_ref[b])
        dl = jnp.exp(bb_ref[b, 0]) * _STRIDE
        dt = jnp.exp(bb_ref[b, 1]) * _STRIDE
        dr = jnp.exp(bb_ref[b, 2]) * _STRIDE
        db = jnp.exp(bb_ref[b, 3]) * _STRIDE

        x1 = jnp.clip(px - dl, 0.0, img_max)
        y1 = jnp.clip(py - dt, 0.0, img_max)
        x2 = jnp.clip(px + dr, 0.0, img_max)
        y2 = jnp.clip(py + db, 0.0, img_max)

        cx = jnp.minimum(dl, dr) / (jnp.maximum(dl, dr) + _EPS)
        cy = jnp.minimum(dt, db) / (jnp.maximum(dt, db) + _EPS)
        ctr = jnp.sqrt(jnp.clip(cx * cy, 0.0, None))

        s = jnp.where(raw > _SCORE_THR, raw * ctr, 0.0)
        s = jnp.where(kf < n_valid, s, 0.0)

        sx1[...] = x1
        sy1[...] = y1
        sx2[...] = x2
        sy2[...] = y2
        sar[...] = jnp.clip(x2 - x1, 0.0, None) * jnp.clip(y2 - y1, 0.0, None)
        ss[...] = s

        swork = s
        cx1 = []
        cy1 = []
        cx2 = []
        cy2 = []
        csc = []
        ckf = []
        for t in range(_TOPK):
            mlane = jnp.max(swork, axis=0, keepdims=True)
            live = mlane > 0.0
            rsel = jnp.min(jnp.where(swork == mlane, rowf, _BIG),
                           axis=0, keepdims=True)
            mask = (rowf == rsel) & live
            cx1.append(jnp.sum(jnp.where(mask, x1, 0.0), axis=0,
                               keepdims=True))
            cy1.append(jnp.sum(jnp.where(mask, y1, 0.0), axis=0,
                               keepdims=True))
            cx2.append(jnp.sum(jnp.where(mask, x2, 0.0), axis=0,
                               keepdims=True))
            cy2.append(jnp.sum(jnp.where(mask, y2, 0.0), axis=0,
                               keepdims=True))
            csc.append(jnp.where(live, mlane, 0.0))
            ckf.append(jnp.where(live, rsel * 128.0 + lanef,
                                 -1.0 - lanef - 128.0 * t))
            swork = jnp.where(mask, 0.0, swork)
        CX1 = jnp.concatenate(cx1, axis=0)
        CY1 = jnp.concatenate(cy1, axis=0)
        CX2 = jnp.concatenate(cx2, axis=0)
        CY2 = jnp.concatenate(cy2, axis=0)
        CAR = (jnp.clip(CX2 - CX1, 0.0, None)
               * jnp.clip(CY2 - CY1, 0.0, None))
        CSC = jnp.concatenate(csc, axis=0)
        CKF = jnp.concatenate(ckf, axis=0)
        compact.append((CX1, CY1, CX2, CY2, CAR, CSC, CKF))
        rest_max.append(jnp.max(swork))

    lane_i = jax.lax.broadcasted_iota(jnp.int32, (1, 128), 1)
    zero = jnp.zeros((1, 128), jnp.float32)

    ms = []
    kos = []
    for b in range(B):
        CSC = compact[b][5]
        M = _allred(CSC, jnp.maximum)
        ms.append(M)
        kos.append(_allred(jnp.where(CSC == M, compact[b][6], _BIG),
                           jnp.minimum))
    accs0 = [[zero] * 5 for _ in range(B)]
    nfs0 = [jnp.zeros((1, 128), jnp.float32) for _ in range(B)]
    cscs0 = [compact[b][5] for b in range(B)]

    def cbody(i, carry):
        ms, kos, cscs, nfs, accs = carry
        n_ms = []
        n_kos = []
        n_cscs = []
        n_nfs = []
        n_accs = []
        for b in range(B):
            CX1, CY1, CX2, CY2, CAR, _, CKF = compact[b]
            M = ms[b]
            KO = kos[b]
            csc = cscs[b]
            mask1 = CKF == KO
            rx1 = _allred(jnp.where(mask1, CX1, 0.0), jnp.add)
            ry1 = _allred(jnp.where(mask1, CY1, 0.0), jnp.add)
            rx2 = _allred(jnp.where(mask1, CX2, 0.0), jnp.add)
            ry2 = _allred(jnp.where(mask1, CY2, 0.0), jnp.add)

            ix1 = jnp.maximum(rx1, CX1)
            iy1 = jnp.maximum(ry1, CY1)
            ix2 = jnp.minimum(rx2, CX2)
            iy2 = jnp.minimum(ry2, CY2)
            inter = (jnp.clip(ix2 - ix1, 0.0, None)
                     * jnp.clip(iy2 - iy1, 0.0, None))
            ba = (jnp.clip(rx2 - rx1, 0.0, None)
                  * jnp.clip(ry2 - ry1, 0.0, None))
            iou = inter / (ba + CAR - inter + _EPS)
            kill = (iou > _IOU_THR) | mask1
            nsc = jnp.where(kill, 0.0, csc)

            NM = _allred(nsc, jnp.maximum)
            n_ms.append(NM)
            n_kos.append(_allred(jnp.where(nsc == NM, CKF, _BIG),
                                 jnp.minimum))
            n_cscs.append(nsc)
            n_nfs.append(jnp.where(M[0:1, :] <= rest_max[b], 1.0, nfs[b]))

            valid = M[0:1, :] > 0.0
            sel = lane_i == i
            vals = (rx1[0:1, :], ry1[0:1, :], rx2[0:1, :], ry2[0:1, :],
                    M[0:1, :])
            n_accs.append([
                jnp.where(sel, jnp.where(valid, v, 0.0), a)
                for v, a in zip(vals, accs[b])])
        return (n_ms, n_kos, n_cscs, n_nfs, n_accs)

    _, _, _, nfs, accs = jax.lax.fori_loop(
        0, _MAX_NUM, cbody, (ms, kos, cscs0, nfs0, accs0), unroll=100)

    def make_fallback(b):
        def fallback():
            sx1, sy1, sx2, sy2, sar, ss = sb[b]
            s0 = ss[...]
            kfv = skf[...]
            m0 = jnp.max(s0)
            idx0 = _argmin_idx(s0, m0, kfv)

            def fbody(i, carry):
                m, idx, sup, faccs = carry
                ii = idx.astype(jnp.int32)
                row = ii >> 7
                lane = ii & 127
                onerow = lane_i == lane
                rx1 = jnp.sum(jnp.where(onerow, sx1[pl.ds(row, 1), :], 0.0))
                ry1 = jnp.sum(jnp.where(onerow, sy1[pl.ds(row, 1), :], 0.0))
                rx2 = jnp.sum(jnp.where(onerow, sx2[pl.ds(row, 1), :], 0.0))
                ry2 = jnp.sum(jnp.where(onerow, sy2[pl.ds(row, 1), :], 0.0))
                x1 = sx1[...]
                y1 = sy1[...]
                x2 = sx2[...]
                y2 = sy2[...]
                area = sar[...]
                kf2 = skf[...]
                ix1 = jnp.maximum(rx1, x1)
                iy1 = jnp.maximum(ry1, y1)
                ix2 = jnp.minimum(rx2, x2)
                iy2 = jnp.minimum(ry2, y2)
                inter = (jnp.clip(ix2 - ix1, 0.0, None)
                         * jnp.clip(iy2 - iy1, 0.0, None))
                ba = (jnp.clip(rx2 - rx1, 0.0, None)
                      * jnp.clip(ry2 - ry1, 0.0, None))
                iou = inter / (ba + area - inter + _EPS)
                kill = (iou > _IOU_THR) | (kf2 == idx)
                sup2 = jnp.where(kill, 0.0, sup)
                nm = jnp.max(sup2)
                nidx = _argmin_idx(sup2, nm, kf2)
                valid = m > 0.0
                sel = lane_i == i
                vals = (rx1, ry1, rx2, ry2, m)
                nfaccs = [jnp.where(sel, jnp.where(valid, v, 0.0), a)
                          for v, a in zip(vals, faccs)]
                return (nm, nidx, sup2, nfaccs)

            _, _, _, faccs = jax.lax.fori_loop(
                0, _MAX_NUM, fbody, (m0, idx0, s0, [zero] * 5))
            return faccs
        return fallback

    for b in range(B):
        need_fb = (nfs[b][0, 0] > 0.0) & (rest_max[b] > 0.0)
        accs_b = jax.lax.cond(need_fb, make_fallback(b),
                              lambda accs_b=accs[b]: accs_b)
        for c in range(5):
            out_ref[b, c:c + 1, :] = accs_b[c]
        out_ref[b, 5:8, :] = jnp.zeros((3, 128), jnp.float32)


@jax.jit
def kernel(cls_score, bbox_pred):
    B, C, H, W = cls_score.shape
    N = H * W
    R = (N + 127) // 128
    R = ((R + 7) // 8) * 8
    NP = R * 128
    img_max = float(H) * _STRIDE

    cls_flat = cls_score.reshape(B, N)
    cls_flat = jnp.pad(cls_flat, ((0, 0), (0, NP - N)), constant_values=-30.0)
    cls_flat = cls_flat.reshape(B, R, 128)

    bb_flat = bbox_pred.reshape(B, 4, N)
    bb_flat = jnp.pad(bb_flat, ((0, 0), (0, 0), (0, NP - N)))
    bb_flat = bb_flat.reshape(B, 4, R, 128)

    k = np.arange(NP)
    ix = (k % W).astype(np.float32)
    iy = (k // W).astype(np.float32)
    px = jnp.asarray(((ix + 0.5) * _STRIDE).reshape(R, 128))
    py = jnp.asarray(((iy + 0.5) * _STRIDE).reshape(R, 128))

    def body(*refs):
        _nms_body(img_max, float(N), B, *refs)

    scratch = [pltpu.VMEM((R, 128), jnp.float32)] * (1 + 6 * B)
    out = pl.pallas_call(
        body,
        out_specs=pl.BlockSpec((B, 8, 128), lambda: (0, 0, 0)),
        out_shape=jax.ShapeDtypeStruct((B, 8, 128), jnp.float32),
        in_specs=[
            pl.BlockSpec((B, R, 128), lambda: (0, 0, 0)),
            pl.BlockSpec((B, 4, R, 128), lambda: (0, 0, 0, 0)),
            pl.BlockSpec((R, 128), lambda: (0, 0)),
            pl.BlockSpec((R, 128), lambda: (0, 0)),
        ],
        scratch_shapes=tuple(scratch),
    )(cls_flat, bb_flat, px, py)

    det = out[:, :5, :_MAX_NUM].transpose(0, 2, 1)
    labels = jnp.zeros((B, _MAX_NUM), jnp.int32)
    return det, labels


# speedup vs baseline: 2.6066x; 2.6066x over previous
import jax
import jax.numpy as jnp
import numpy as np
from jax.experimental import pallas as pl
from jax.experimental.pallas import tpu as pltpu

_SCORE_THR = 0.05
_IOU_THR = 0.5
_MAX_NUM = 100
_STRIDE = 8.0
_EPS = 1e-6
_BIG = 3.0e7
_TOPK = 8


def _argmin_idx(s, m, kf):
    return jnp.min(jnp.where(s == m, kf, _BIG))


def _nms_body(img_max, n_valid, B, cls_ref, bb_ref, px_ref, py_ref,
              out_ref, *scratch):
    skf = scratch[0]
    sb = [scratch[1 + 6 * b:1 + 6 * (b + 1)] for b in range(B)]

    shape = px_ref.shape
    row_i = jax.lax.broadcasted_iota(jnp.int32, shape, 0)
    col_i = jax.lax.broadcasted_iota(jnp.int32, shape, 1)
    kf = (row_i * 128 + col_i).astype(jnp.float32)
    rowf = row_i.astype(jnp.float32)
    skf[...] = kf
    lanef = jax.lax.broadcasted_iota(jnp.int32, (1, 128), 1).astype(jnp.float32)
    px = px_ref[...]
    py = py_ref[...]

    compact = []
    rest_max = []
    for b in range(B):
        sx1, sy1, sx2, sy2, sar, ss = sb[b]
        raw = jax.nn.sigmoid(cls_ref[b])
        dl = jnp.exp(bb_ref[b, 0]) * _STRIDE
        dt = jnp.exp(bb_ref[b, 1]) * _STRIDE
        dr = jnp.exp(bb_ref[b, 2]) * _STRIDE
        db = jnp.exp(bb_ref[b, 3]) * _STRIDE

        x1 = jnp.clip(px - dl, 0.0, img_max)
        y1 = jnp.clip(py - dt, 0.0, img_max)
        x2 = jnp.clip(px + dr, 0.0, img_max)
        y2 = jnp.clip(py + db, 0.0, img_max)

        cx = jnp.minimum(dl, dr) / (jnp.maximum(dl, dr) + _EPS)
        cy = jnp.minimum(dt, db) / (jnp.maximum(dt, db) + _EPS)
        ctr = jnp.sqrt(jnp.clip(cx * cy, 0.0, None))

        s = jnp.where(raw > _SCORE_THR, raw * ctr, 0.0)
        s = jnp.where(kf < n_valid, s, 0.0)

        sx1[...] = x1
        sy1[...] = y1
        sx2[...] = x2
        sy2[...] = y2
        sar[...] = jnp.clip(x2 - x1, 0.0, None) * jnp.clip(y2 - y1, 0.0, None)
        ss[...] = s

        swork = s
        cx1 = []
        cy1 = []
        cx2 = []
        cy2 = []
        csc = []
        ckf = []
        for t in range(_TOPK):
            mlane = jnp.max(swork, axis=0, keepdims=True)
            live = mlane > 0.0
            rsel = jnp.min(jnp.where(swork == mlane, rowf, _BIG),
                           axis=0, keepdims=True)
            mask = (rowf == rsel) & live
            cx1.append(jnp.sum(jnp.where(mask, x1, 0.0), axis=0,
                               keepdims=True))
            cy1.append(jnp.sum(jnp.where(mask, y1, 0.0), axis=0,
                               keepdims=True))
            cx2.append(jnp.sum(jnp.where(mask, x2, 0.0), axis=0,
                               keepdims=True))
            cy2.append(jnp.sum(jnp.where(mask, y2, 0.0), axis=0,
                               keepdims=True))
            csc.append(jnp.where(live, mlane, 0.0))
            ckf.append(jnp.where(live, rsel * 128.0 + lanef,
                                 -1.0 - lanef - 128.0 * t))
            swork = jnp.where(mask, 0.0, swork)
        CX1 = jnp.concatenate(cx1, axis=0)
        CY1 = jnp.concatenate(cy1, axis=0)
        CX2 = jnp.concatenate(cx2, axis=0)
        CY2 = jnp.concatenate(cy2, axis=0)
        CAR = (jnp.clip(CX2 - CX1, 0.0, None)
               * jnp.clip(CY2 - CY1, 0.0, None))
        CSC = jnp.concatenate(csc, axis=0)
        CKF = jnp.concatenate(ckf, axis=0)
        compact.append((CX1, CY1, CX2, CY2, CAR, CSC, CKF))
        rest_max.append(jnp.max(swork))

    lane_i = jax.lax.broadcasted_iota(jnp.int32, (1, 128), 1)
    zero = jnp.zeros((1, 128), jnp.float32)

    ms = []
    kos = []
    for b in range(B):
        CSC = compact[b][5]
        m = jnp.max(CSC)
        ms.append(m)
        kos.append(_argmin_idx(CSC, m, compact[b][6]))
    accs0 = [[zero] * 5 for _ in range(B)]
    nfs0 = [jnp.zeros((), jnp.bool_) for _ in range(B)]
    cscs0 = [compact[b][5] for b in range(B)]

    def cbody(i, carry):
        ms, kos, cscs, nfs, accs = carry
        n_ms = []
        n_kos = []
        n_cscs = []
        n_nfs = []
        n_accs = []
        for b in range(B):
            CX1, CY1, CX2, CY2, CAR, _, CKF = compact[b]
            m = ms[b]
            ko = kos[b]
            csc = cscs[b]
            mask1 = CKF == ko
            rx1 = jnp.sum(jnp.where(mask1, CX1, 0.0))
            ry1 = jnp.sum(jnp.where(mask1, CY1, 0.0))
            rx2 = jnp.sum(jnp.where(mask1, CX2, 0.0))
            ry2 = jnp.sum(jnp.where(mask1, CY2, 0.0))

            ix1 = jnp.maximum(rx1, CX1)
            iy1 = jnp.maximum(ry1, CY1)
            ix2 = jnp.minimum(rx2, CX2)
            iy2 = jnp.minimum(ry2, CY2)
            inter = (jnp.clip(ix2 - ix1, 0.0, None)
                     * jnp.clip(iy2 - iy1, 0.0, None))
            ba = (jnp.clip(rx2 - rx1, 0.0, None)
                  * jnp.clip(ry2 - ry1, 0.0, None))
            iou = inter / (ba + CAR - inter + _EPS)
            kill = (iou > _IOU_THR) | mask1
            nsc = jnp.where(kill, 0.0, csc)

            nm = jnp.max(nsc)
            n_ms.append(nm)
            n_kos.append(_argmin_idx(nsc, nm, CKF))
            n_cscs.append(nsc)
            n_nfs.append(nfs[b] | (m <= rest_max[b]))

            valid = m > 0.0
            sel = lane_i == i
            vals = (rx1, ry1, rx2, ry2, m)
            n_accs.append([
                jnp.where(sel, jnp.where(valid, v, 0.0), a)
                for v, a in zip(vals, accs[b])])
        return (n_ms, n_kos, n_cscs, n_nfs, n_accs)

    _, _, _, nfs, accs = jax.lax.fori_loop(
        0, _MAX_NUM, cbody, (ms, kos, cscs0, nfs0, accs0), unroll=100)

    def make_fallback(b):
        def fallback():
            sx1, sy1, sx2, sy2, sar, ss = sb[b]
            s0 = ss[...]
            kfv = skf[...]
            m0 = jnp.max(s0)
            idx0 = _argmin_idx(s0, m0, kfv)

            def fbody(i, carry):
                m, idx, sup, faccs = carry
                ii = idx.astype(jnp.int32)
                row = ii >> 7
                lane = ii & 127
                onerow = lane_i == lane
                rx1 = jnp.sum(jnp.where(onerow, sx1[pl.ds(row, 1), :], 0.0))
                ry1 = jnp.sum(jnp.where(onerow, sy1[pl.ds(row, 1), :], 0.0))
                rx2 = jnp.sum(jnp.where(onerow, sx2[pl.ds(row, 1), :], 0.0))
                ry2 = jnp.sum(jnp.where(onerow, sy2[pl.ds(row, 1), :], 0.0))
                x1 = sx1[...]
                y1 = sy1[...]
                x2 = sx2[...]
                y2 = sy2[...]
                area = sar[...]
                kf2 = skf[...]
                ix1 = jnp.maximum(rx1, x1)
                iy1 = jnp.maximum(ry1, y1)
                ix2 = jnp.minimum(rx2, x2)
                iy2 = jnp.minimum(ry2, y2)
                inter = (jnp.clip(ix2 - ix1, 0.0, None)
                         * jnp.clip(iy2 - iy1, 0.0, None))
                ba = (jnp.clip(rx2 - rx1, 0.0, None)
                      * jnp.clip(ry2 - ry1, 0.0, None))
                iou = inter / (ba + area - inter + _EPS)
                kill = (iou > _IOU_THR) | (kf2 == idx)
                sup2 = jnp.where(kill, 0.0, sup)
                nm = jnp.max(sup2)
                nidx = _argmin_idx(sup2, nm, kf2)
                valid = m > 0.0
                sel = lane_i == i
                vals = (rx1, ry1, rx2, ry2, m)
                nfaccs = [jnp.where(sel, jnp.where(valid, v, 0.0), a)
                          for v, a in zip(vals, faccs)]
                return (nm, nidx, sup2, nfaccs)

            _, _, _, faccs = jax.lax.fori_loop(
                0, _MAX_NUM, fbody, (m0, idx0, s0, [zero] * 5))
            return faccs
        return fallback

    for b in range(B):
        need_fb = nfs[b] & (rest_max[b] > 0.0)
        accs_b = jax.lax.cond(need_fb, make_fallback(b),
                              lambda accs_b=accs[b]: accs_b)
        for c in range(5):
            out_ref[b, c:c + 1, :] = accs_b[c]
        out_ref[b, 5:8, :] = jnp.zeros((3, 128), jnp.float32)


@jax.jit
def kernel(cls_score, bbox_pred):
    B, C, H, W = cls_score.shape
    N = H * W
    R = (N + 127) // 128
    R = ((R + 7) // 8) * 8
    NP = R * 128
    img_max = float(H) * _STRIDE

    cls_flat = cls_score.reshape(B, N)
    cls_flat = jnp.pad(cls_flat, ((0, 0), (0, NP - N)), constant_values=-30.0)
    cls_flat = cls_flat.reshape(B, R, 128)

    bb_flat = bbox_pred.reshape(B, 4, N)
    bb_flat = jnp.pad(bb_flat, ((0, 0), (0, 0), (0, NP - N)))
    bb_flat = bb_flat.reshape(B, 4, R, 128)

    k = np.arange(NP)
    ix = (k % W).astype(np.float32)
    iy = (k // W).astype(np.float32)
    px = jnp.asarray(((ix + 0.5) * _STRIDE).reshape(R, 128))
    py = jnp.asarray(((iy + 0.5) * _STRIDE).reshape(R, 128))

    def body(*refs):
        _nms_body(img_max, float(N), B, *refs)

    scratch = [pltpu.VMEM((R, 128), jnp.float32)] * (1 + 6 * B)
    out = pl.pallas_call(
        body,
        out_specs=pl.BlockSpec((B, 8, 128), lambda: (0, 0, 0)),
        out_shape=jax.ShapeDtypeStruct((B, 8, 128), jnp.float32),
        in_specs=[
            pl.BlockSpec((B, R, 128), lambda: (0, 0, 0)),
            pl.BlockSpec((B, 4, R, 128), lambda: (0, 0, 0, 0)),
            pl.BlockSpec((R, 128), lambda: (0, 0)),
            pl.BlockSpec((R, 128), lambda: (0, 0)),
        ],
        scratch_shapes=tuple(scratch),
    )(cls_flat, bb_flat, px, py)

    det = out[:, :5, :_MAX_NUM].transpose(0, 2, 1)
    labels = jnp.zeros((B, _MAX_NUM), jnp.int32)
    return det, labels
